# Initial kernel scaffold; baseline (speedup 1.0000x reference)
import jax
import jax.numpy as jnp
from jax.experimental import pallas as pl


def kernel(loc_data, conf_data, loc_data_r, conf_data_r, priors, targets):
    raise NotImplementedError("write your pallas kernel here")


# speedup vs baseline: 12.5384x; 12.5384x over previous
import jax
import jax.numpy as jnp
from jax.experimental import pallas as pl
from jax.experimental.pallas import tpu as pltpu

NUM_CLASSES = 21
THRESHOLD = 0.5
NEGPOS_RATIO = 3
V0 = 0.1
V1 = 0.2
BATCH = 16
P = 20000
NOBJ = 12


def _huber(d):
    ad = jnp.abs(d)
    return jnp.where(ad < 1.0, 0.5 * ad * ad, ad - 0.5)


def _match_branch(tb, bcx, bcy, bw, bh, binary):
    tx1 = tb[:, 0:1]
    ty1 = tb[:, 1:2]
    tx2 = tb[:, 2:3]
    ty2 = tb[:, 3:4]
    lab = tb[:, 4:5]
    area_t = (tx2 - tx1) * (ty2 - ty1)

    px1 = bcx - bw * 0.5
    py1 = bcy - bh * 0.5
    px2 = bcx + bw * 0.5
    py2 = bcy + bh * 0.5
    area_p = (px2 - px1) * (py2 - py1)

    ix = jnp.clip(jnp.minimum(tx2, px2) - jnp.maximum(tx1, px1), 0.0, None)
    iy = jnp.clip(jnp.minimum(ty2, py2) - jnp.maximum(ty1, py1), 0.0, None)
    inter = ix * iy
    union = area_t + area_p - inter
    ov = inter / jnp.maximum(union, 1e-12)

    t_iota = jax.lax.broadcasted_iota(jnp.int32, (NOBJ, P), 0)
    j_iota = jax.lax.broadcasted_iota(jnp.int32, (NOBJ, P), 1)

    bto = jnp.max(ov, axis=0, keepdims=True)
    bti = jnp.min(jnp.where(ov == bto, t_iota, NOBJ), axis=0, keepdims=True)

    rowmax = jnp.max(ov, axis=1, keepdims=True)
    bpi = jnp.min(jnp.where(ov == rowmax, j_iota, P), axis=1, keepdims=True)

    fmask = j_iota == bpi
    forced = jnp.max(fmask.astype(jnp.int32), axis=0, keepdims=True) > 0
    bti_forced = jnp.max(jnp.where(fmask, t_iota, -1), axis=0, keepdims=True)
    bti = jnp.where(forced, bti_forced, bti)
    bto = jnp.where(forced, 2.0, bto)

    teq = t_iota == bti
    mx1 = jnp.sum(jnp.where(teq, tx1, 0.0), axis=0, keepdims=True)
    my1 = jnp.sum(jnp.where(teq, ty1, 0.0), axis=0, keepdims=True)
    mx2 = jnp.sum(jnp.where(teq, tx2, 0.0), axis=0, keepdims=True)
    my2 = jnp.sum(jnp.where(teq, ty2, 0.0), axis=0, keepdims=True)

    if binary:
        conf = jnp.where(bto < THRESHOLD, 0.0, 1.0)
    else:
        labsel = jnp.sum(jnp.where(teq, lab, 0.0), axis=0, keepdims=True)
        conf = jnp.where(bto < THRESHOLD, 0.0, labsel + 1.0)

    pw_ = jnp.maximum(bw, 1e-12)
    ph_ = jnp.maximum(bh, 1e-12)
    gcx = ((mx1 + mx2) * 0.5 - bcx) / (V0 * pw_)
    gcy = ((my1 + my2) * 0.5 - bcy) / (V0 * ph_)
    gw = jnp.log(jnp.maximum((mx2 - mx1) / pw_, 1e-12)) / V1
    gh = jnp.log(jnp.maximum((my2 - my1) / ph_, 1e-12)) / V1

    pos = conf > 0.0
    return (gcx, gcy, gw, gh), conf, pos


def _stepk(targets_ref, priors_ref, loc_ref, conf_ref, locr_ref, confr_ref,
           cem1_ref, cemr_ref, cer_ref, posr_ref, stats_ref):
    tb = targets_ref[0]
    pr = priors_ref[:, :]
    pcx, pcy, pw, ph = pr[0:1], pr[1:2], pr[2:3], pr[3:4]

    ld = loc_ref[0]
    cd = conf_ref[0]
    ldr = locr_ref[0]
    cdr = confr_ref[0]

    lt1, conf1, pos1 = _match_branch(tb, pcx, pcy, pw, ph, True)
    ll_b = jnp.sum(jnp.where(pos1,
                             _huber(ld[0:1] - lt1[0]) + _huber(ld[1:2] - lt1[1])
                             + _huber(ld[2:3] - lt1[2]) + _huber(ld[3:4] - lt1[3]),
                             0.0))

    x0, x1 = cd[0:1], cd[1:2]
    m = jnp.maximum(x0, x1)
    e0 = jnp.exp(x0 - m)
    e1 = jnp.exp(x1 - m)
    lse = jnp.log(e0 + e1) + m
    ce1 = lse - jnp.where(pos1, x1, x0)
    cem1 = jnp.where(pos1, 0.0, ce1)
    lcpos1_b = jnp.sum(jnp.where(pos1, ce1, 0.0))
    refined = (e0 / (e0 + e1)) > 0.99

    dcx = pcx + ld[0:1] * (V0 * pw)
    dcy = pcy + ld[1:2] * (V0 * ph)
    dw = pw * jnp.exp(ld[2:3] * V1)
    dh = ph * jnp.exp(ld[3:4] * V1)
    dcx = jnp.clip(dcx, 0.0, 1.0)
    dcy = jnp.clip(dcy, 0.0, 1.0)
    dw = jnp.clip(dw, 0.0, 1.0)
    dh = jnp.clip(dh, 0.0, 1.0)

    ltr, confr, posr = _match_branch(tb, dcx, dcy, dw, dh, False)
    llr_b = jnp.sum(jnp.where(posr,
                              _huber(ldr[0:1] - ltr[0]) + _huber(ldr[1:2] - ltr[1])
                              + _huber(ldr[2:3] - ltr[2]) + _huber(ldr[3:4] - ltr[3]),
                              0.0))

    mr = jnp.max(cdr, axis=0, keepdims=True)
    exr = jnp.exp(cdr - mr)
    lser = jnp.log(jnp.sum(exr, axis=0, keepdims=True)) + mr
    c_iota = jax.lax.broadcasted_iota(jnp.int32, (NUM_CLASSES, P), 0)
    cfr_int = confr.astype(jnp.int32)
    selv = jnp.sum(jnp.where(c_iota == cfr_int, cdr, 0.0), axis=0, keepdims=True)
    cer = lser - selv
    cemr = jnp.where(posr | refined, 0.0, cer)
    lcposr_b = jnp.sum(jnp.where(posr, cer, 0.0))

    np1 = jnp.sum(pos1.astype(jnp.float32))
    npr = jnp.sum(posr.astype(jnp.float32))

    cem1_ref[0] = cem1
    cemr_ref[0] = cemr
    cer_ref[0] = cer
    posr_ref[0] = posr.astype(jnp.float32)

    li = jax.lax.broadcasted_iota(jnp.int32, (1, 128), 1)
    stats = (jnp.where(li == 0, ll_b, 0.0) + jnp.where(li == 1, lcpos1_b, 0.0)
             + jnp.where(li == 2, llr_b, 0.0) + jnp.where(li == 3, lcposr_b, 0.0)
             + jnp.where(li == 4, np1, 0.0) + jnp.where(li == 5, npr, 0.0))
    stats_ref[0] = stats


def _topk_sum(bits, vals, k):
    lo = jnp.zeros((BATCH, 1), jnp.int32)
    hi = jnp.max(bits, axis=1, keepdims=True)

    def body(_, lh):
        lo, hi = lh
        mid = lo + jax.lax.shift_right_logical(hi - lo + 1, 1)
        cnt = jnp.sum((bits >= mid).astype(jnp.int32), axis=1, keepdims=True)
        ok = cnt >= k
        return jnp.where(ok, mid, lo), jnp.where(ok, hi, mid - 1)

    lo, hi = jax.lax.fori_loop(0, 31, body, (lo, hi))
    gt = bits > lo
    cnt_gt = jnp.sum(gt.astype(jnp.int32), axis=1, keepdims=True)
    sum_gt = jnp.sum(jnp.where(gt, vals, 0.0), axis=1, keepdims=True)
    need = k - cnt_gt
    return lo, cnt_gt, sum_gt, need


def _minek(cem1_ref, cemr_ref, cer_ref, posr_ref, stats_ref,
           o1_ref, o2_ref, o3_ref, o4_ref):
    stats = stats_ref[:, :]
    ll = jnp.sum(stats[:, 0:1])
    lcpos1 = jnp.sum(stats[:, 1:2])
    llr = jnp.sum(stats[:, 2:3])
    lcposr = jnp.sum(stats[:, 3:4])
    np1 = stats[:, 4:5]
    npr = stats[:, 5:6]
    n = jnp.sum(np1)
    nr = jnp.sum(npr)

    cem1 = cem1_ref[:, :]
    cemr = cemr_ref[:, :]
    cer = cer_ref[:, :]
    posr = posr_ref[:, :]

    k1 = jnp.minimum(np1 * NEGPOS_RATIO, float(P - 1)).astype(jnp.int32)
    kr = jnp.minimum(npr * NEGPOS_RATIO, float(P - 1)).astype(jnp.int32)

    bits1 = jax.lax.bitcast_convert_type(cem1, jnp.int32)
    bitsr = jax.lax.bitcast_convert_type(cemr, jnp.int32)

    lo1, _, sum_gt1, need1 = _topk_sum(bits1, cem1, k1)
    t1 = jax.lax.bitcast_convert_type(lo1, jnp.float32)
    loss_c = lcpos1 + jnp.sum(sum_gt1 + t1 * need1.astype(jnp.float32))

    lor, _, sum_gtr, needr = _topk_sum(bitsr, cemr, kr)
    tr = jax.lax.bitcast_convert_type(lor, jnp.float32)
    loss_cr = lcposr + jnp.sum(sum_gtr + tr * needr.astype(jnp.float32))

    need0 = jnp.where(lor == 0, needr, 0)
    zeros = cemr == 0.0
    j_iota = jax.lax.broadcasted_iota(jnp.int32, (BATCH, P), 1)

    def body2(_, lh):
        lo, hi = lh
        mid = jax.lax.shift_right_logical(lo + hi, 1)
        f = jnp.sum((zeros & (j_iota < mid)).astype(jnp.int32), axis=1,
                    keepdims=True)
        ok = f >= need0
        return jnp.where(ok, lo, mid + 1), jnp.where(ok, mid, hi)

    lo2 = jnp.zeros((BATCH, 1), jnp.int32)
    hi2 = jnp.full((BATCH, 1), P, jnp.int32)
    _, istar = jax.lax.fori_loop(0, 15, body2, (lo2, hi2))
    pick = zeros & (j_iota < istar)
    corr = jnp.sum(jnp.where(pick & (posr == 0.0), cer, 0.0))
    loss_cr = loss_cr + corr

    o1_ref[:, :] = (ll / n).reshape(1, 1)
    o2_ref[:, :] = (loss_c / n).reshape(1, 1)
    o3_ref[:, :] = (llr / nr).reshape(1, 1)
    o4_ref[:, :] = (loss_cr / nr).reshape(1, 1)


def kernel(loc_data, conf_data, loc_data_r, conf_data_r, priors, targets):
    loc_t = jnp.transpose(loc_data, (0, 2, 1))
    conf_t = jnp.transpose(conf_data, (0, 2, 1))
    locr_t = jnp.transpose(loc_data_r, (0, 2, 1))
    confr_t = jnp.transpose(conf_data_r, (0, 2, 1))
    pri_t = jnp.transpose(priors, (1, 0))

    row = jax.ShapeDtypeStruct((BATCH, 1, P), jnp.float32)
    stats_s = jax.ShapeDtypeStruct((BATCH, 1, 128), jnp.float32)

    cem1, cemr, cer, posr, stats = pl.pallas_call(
        _stepk,
        grid=(BATCH,),
        in_specs=[
            pl.BlockSpec((1, NOBJ, 5), lambda b: (b, 0, 0)),
            pl.BlockSpec((4, P), lambda b: (0, 0)),
            pl.BlockSpec((1, 4, P), lambda b: (b, 0, 0)),
            pl.BlockSpec((1, 2, P), lambda b: (b, 0, 0)),
            pl.BlockSpec((1, 4, P), lambda b: (b, 0, 0)),
            pl.BlockSpec((1, NUM_CLASSES, P), lambda b: (b, 0, 0)),
        ],
        out_specs=[
            pl.BlockSpec((1, 1, P), lambda b: (b, 0, 0)),
            pl.BlockSpec((1, 1, P), lambda b: (b, 0, 0)),
            pl.BlockSpec((1, 1, P), lambda b: (b, 0, 0)),
            pl.BlockSpec((1, 1, P), lambda b: (b, 0, 0)),
            pl.BlockSpec((1, 1, 128), lambda b: (b, 0, 0)),
        ],
        out_shape=[row, row, row, row, stats_s],
    )(targets, pri_t, loc_t, conf_t, locr_t, confr_t)

    cem1 = cem1.reshape(BATCH, P)
    cemr = cemr.reshape(BATCH, P)
    cer = cer.reshape(BATCH, P)
    posr = posr.reshape(BATCH, P)
    stats = stats.reshape(BATCH, 128)

    sc = jax.ShapeDtypeStruct((1, 1), jnp.float32)
    o1, o2, o3, o4 = pl.pallas_call(
        _minek,
        out_shape=[sc, sc, sc, sc],
    )(cem1, cemr, cer, posr, stats)

    return (o1.reshape(()), o2.reshape(()), o3.reshape(()), o4.reshape(()))


# speedup vs baseline: 19.9971x; 1.5949x over previous
import jax
import jax.numpy as jnp
from jax.experimental import pallas as pl
from jax.experimental.pallas import tpu as pltpu

NUM_CLASSES = 21
THRESHOLD = 0.5
NEGPOS_RATIO = 3
V0 = 0.1
V1 = 0.2
BATCH = 16
P = 20000
R = 8
Q = P // R
NOBJ = 12


def _huber(d):
    ad = jnp.abs(d)
    return jnp.where(ad < 1.0, 0.5 * ad * ad, ad - 0.5)


def _match_branch(tb, bcx, bcy, bw, bh, binary):
    tx1 = tb[:, 0:1][:, :, None]
    ty1 = tb[:, 1:2][:, :, None]
    tx2 = tb[:, 2:3][:, :, None]
    ty2 = tb[:, 3:4][:, :, None]
    lab = tb[:, 4:5][:, :, None]
    area_t = (tx2 - tx1) * (ty2 - ty1)

    px1 = (bcx - bw * 0.5)[None]
    py1 = (bcy - bh * 0.5)[None]
    px2 = (bcx + bw * 0.5)[None]
    py2 = (bcy + bh * 0.5)[None]
    area_p = (px2 - px1) * (py2 - py1)

    ix = jnp.maximum(jnp.minimum(tx2, px2) - jnp.maximum(tx1, px1), 0.0)
    iy = jnp.maximum(jnp.minimum(ty2, py2) - jnp.maximum(ty1, py1), 0.0)
    inter = ix * iy
    union = area_t + area_p - inter
    ov = inter / jnp.maximum(union, 1e-12)

    t_iota = jax.lax.broadcasted_iota(jnp.int32, (NOBJ, 1, 1), 0)
    pidx = (jax.lax.broadcasted_iota(jnp.int32, (R, Q), 0) * Q
            + jax.lax.broadcasted_iota(jnp.int32, (R, Q), 1))[None]

    bto3 = jnp.max(ov, axis=0, keepdims=True)
    bti = jnp.min(jnp.where(ov == bto3, t_iota, NOBJ), axis=0)

    rowmax = jnp.max(ov, axis=(1, 2), keepdims=True)
    bpi = jnp.min(jnp.where(ov == rowmax, pidx, P), axis=(1, 2),
                  keepdims=True)

    fmask = pidx == bpi
    forced = jnp.max(fmask.astype(jnp.int32), axis=0) > 0
    bti_forced = jnp.max(jnp.where(fmask, t_iota, -1), axis=0)
    bti = jnp.where(forced, bti_forced, bti)
    bto = jnp.where(forced, 2.0, bto3[0])

    teq = t_iota == bti[None]
    mx1 = jnp.sum(jnp.where(teq, tx1, 0.0), axis=0)
    my1 = jnp.sum(jnp.where(teq, ty1, 0.0), axis=0)
    mx2 = jnp.sum(jnp.where(teq, tx2, 0.0), axis=0)
    my2 = jnp.sum(jnp.where(teq, ty2, 0.0), axis=0)

    if binary:
        conf = jnp.where(bto < THRESHOLD, 0.0, 1.0)
    else:
        labsel = jnp.sum(jnp.where(teq, lab, 0.0), axis=0)
        conf = jnp.where(bto < THRESHOLD, 0.0, labsel + 1.0)

    pw_ = jnp.maximum(bw, 1e-12)
    ph_ = jnp.maximum(bh, 1e-12)
    gcx = ((mx1 + mx2) * 0.5 - bcx) / (V0 * pw_)
    gcy = ((my1 + my2) * 0.5 - bcy) / (V0 * ph_)
    gw = jnp.log(jnp.maximum((mx2 - mx1) / pw_, 1e-12)) / V1
    gh = jnp.log(jnp.maximum((my2 - my1) / ph_, 1e-12)) / V1

    pos = conf > 0.0
    return (gcx, gcy, gw, gh), conf, pos


def _stepk(targets_ref, priors_ref, loc_ref, conf_ref, locr_ref, confr_ref,
           cem1_ref, cemr_ref, cer_ref, posr_ref, stats_ref):
    tb = targets_ref[0]
    pr = priors_ref[:, :].reshape(4, R, Q)
    pcx, pcy, pw, ph = pr[0], pr[1], pr[2], pr[3]

    ld = loc_ref[0].reshape(4, R, Q)
    cd = conf_ref[0].reshape(2, R, Q)
    ldr = locr_ref[0].reshape(4, R, Q)
    cdr = confr_ref[0].reshape(NUM_CLASSES, R, Q)

    lt1, conf1, pos1 = _match_branch(tb, pcx, pcy, pw, ph, True)
    ll_b = jnp.sum(jnp.where(pos1,
                             _huber(ld[0] - lt1[0]) + _huber(ld[1] - lt1[1])
                             + _huber(ld[2] - lt1[2]) + _huber(ld[3] - lt1[3]),
                             0.0))

    x0, x1 = cd[0], cd[1]
    m = jnp.maximum(x0, x1)
    e0 = jnp.exp(x0 - m)
    e1 = jnp.exp(x1 - m)
    lse = jnp.log(e0 + e1) + m
    ce1 = lse - jnp.where(pos1, x1, x0)
    cem1 = jnp.where(pos1, 0.0, ce1)
    lcpos1_b = jnp.sum(jnp.where(pos1, ce1, 0.0))
    refined = (e0 / (e0 + e1)) > 0.99

    dcx = jnp.clip(pcx + ld[0] * (V0 * pw), 0.0, 1.0)
    dcy = jnp.clip(pcy + ld[1] * (V0 * ph), 0.0, 1.0)
    dw = jnp.clip(pw * jnp.exp(ld[2] * V1), 0.0, 1.0)
    dh = jnp.clip(ph * jnp.exp(ld[3] * V1), 0.0, 1.0)

    ltr, confr, posr = _match_branch(tb, dcx, dcy, dw, dh, False)
    llr_b = jnp.sum(jnp.where(posr,
                              _huber(ldr[0] - ltr[0]) + _huber(ldr[1] - ltr[1])
                              + _huber(ldr[2] - ltr[2]) + _huber(ldr[3] - ltr[3]),
                              0.0))

    mr = jnp.max(cdr, axis=0, keepdims=True)
    exr = jnp.exp(cdr - mr)
    lser = jnp.log(jnp.sum(exr, axis=0)) + mr[0]
    c_iota = jax.lax.broadcasted_iota(jnp.int32, (NUM_CLASSES, 1, 1), 0)
    cfr_int = confr.astype(jnp.int32)[None]
    selv = jnp.sum(jnp.where(c_iota == cfr_int, cdr, 0.0), axis=0)
    cer = lser - selv
    cemr = jnp.where(posr | refined, 0.0, cer)
    lcposr_b = jnp.sum(jnp.where(posr, cer, 0.0))

    np1 = jnp.sum(pos1.astype(jnp.float32))
    npr = jnp.sum(posr.astype(jnp.float32))

    cem1_ref[0] = cem1
    cemr_ref[0] = cemr
    cer_ref[0] = cer
    posr_ref[0] = posr.astype(jnp.float32)

    li = jax.lax.broadcasted_iota(jnp.int32, (1, 128), 1)
    stats = (jnp.where(li == 0, ll_b, 0.0) + jnp.where(li == 1, lcpos1_b, 0.0)
             + jnp.where(li == 2, llr_b, 0.0) + jnp.where(li == 3, lcposr_b, 0.0)
             + jnp.where(li == 4, np1, 0.0) + jnp.where(li == 5, npr, 0.0))
    stats_ref[0] = stats


def _topk_sum(bits, vals, k):
    lo = jnp.zeros((BATCH, 1), jnp.int32)
    hi = jnp.max(bits, axis=1, keepdims=True)

    def body(_, lh):
        lo, hi = lh
        mid = lo + jax.lax.shift_right_logical(hi - lo + 1, 1)
        cnt = jnp.sum((bits >= mid).astype(jnp.int32), axis=1, keepdims=True)
        ok = cnt >= k
        return jnp.where(ok, mid, lo), jnp.where(ok, hi, mid - 1)

    lo, hi = jax.lax.fori_loop(0, 31, body, (lo, hi))
    gt = bits > lo
    cnt_gt = jnp.sum(gt.astype(jnp.int32), axis=1, keepdims=True)
    sum_gt = jnp.sum(jnp.where(gt, vals, 0.0), axis=1, keepdims=True)
    need = k - cnt_gt
    return lo, sum_gt, need


def _minek(cem1_ref, cemr_ref, cer_ref, posr_ref, stats_ref,
           o1_ref, o2_ref, o3_ref, o4_ref):
    stats = stats_ref[:, :]
    ll = jnp.sum(stats[:, 0:1])
    lcpos1 = jnp.sum(stats[:, 1:2])
    llr = jnp.sum(stats[:, 2:3])
    lcposr = jnp.sum(stats[:, 3:4])
    np1 = stats[:, 4:5]
    npr = stats[:, 5:6]
    n = jnp.sum(np1)
    nr = jnp.sum(npr)

    cem1 = cem1_ref[:, :]
    cemr = cemr_ref[:, :]
    cer = cer_ref[:, :]
    posr = posr_ref[:, :]

    k1 = jnp.minimum(np1 * NEGPOS_RATIO, float(P - 1)).astype(jnp.int32)
    kr = jnp.minimum(npr * NEGPOS_RATIO, float(P - 1)).astype(jnp.int32)

    bits1 = jax.lax.bitcast_convert_type(cem1, jnp.int32)
    bitsr = jax.lax.bitcast_convert_type(cemr, jnp.int32)

    lo1, sum_gt1, need1 = _topk_sum(bits1, cem1, k1)
    t1 = jax.lax.bitcast_convert_type(lo1, jnp.float32)
    loss_c = lcpos1 + jnp.sum(sum_gt1 + t1 * need1.astype(jnp.float32))

    lor, sum_gtr, needr = _topk_sum(bitsr, cemr, kr)
    tr = jax.lax.bitcast_convert_type(lor, jnp.float32)
    loss_cr = lcposr + jnp.sum(sum_gtr + tr * needr.astype(jnp.float32))

    need0 = jnp.where(lor == 0, needr, 0)
    zeros = cemr == 0.0
    j_iota = jax.lax.broadcasted_iota(jnp.int32, (BATCH, P), 1)

    def body2(_, lh):
        lo, hi = lh
        mid = jax.lax.shift_right_logical(lo + hi, 1)
        f = jnp.sum((zeros & (j_iota < mid)).astype(jnp.int32), axis=1,
                    keepdims=True)
        ok = f >= need0
        return jnp.where(ok, lo, mid + 1), jnp.where(ok, mid, hi)

    lo2 = jnp.zeros((BATCH, 1), jnp.int32)
    hi2 = jnp.full((BATCH, 1), P, jnp.int32)
    _, istar = jax.lax.fori_loop(0, 15, body2, (lo2, hi2))
    pick = zeros & (j_iota < istar)
    corr = jnp.sum(jnp.where(pick & (posr == 0.0), cer, 0.0))
    loss_cr = loss_cr + corr

    o1_ref[:, :] = (ll / n).reshape(1, 1)
    o2_ref[:, :] = (loss_c / n).reshape(1, 1)
    o3_ref[:, :] = (llr / nr).reshape(1, 1)
    o4_ref[:, :] = (loss_cr / nr).reshape(1, 1)


def kernel(loc_data, conf_data, loc_data_r, conf_data_r, priors, targets):
    loc_t = jnp.transpose(loc_data, (0, 2, 1)).reshape(BATCH, 4 * R, Q)
    conf_t = jnp.transpose(conf_data, (0, 2, 1)).reshape(BATCH, 2 * R, Q)
    locr_t = jnp.transpose(loc_data_r, (0, 2, 1)).reshape(BATCH, 4 * R, Q)
    confr_t = jnp.transpose(conf_data_r, (0, 2, 1)).reshape(
        BATCH, NUM_CLASSES * R, Q)
    pri_t = jnp.transpose(priors, (1, 0)).reshape(4 * R, Q)

    row = jax.ShapeDtypeStruct((BATCH, R, Q), jnp.float32)
    stats_s = jax.ShapeDtypeStruct((BATCH, 1, 128), jnp.float32)

    cem1, cemr, cer, posr, stats = pl.pallas_call(
        _stepk,
        grid=(BATCH,),
        in_specs=[
            pl.BlockSpec((1, NOBJ, 5), lambda b: (b, 0, 0)),
            pl.BlockSpec((4 * R, Q), lambda b: (0, 0)),
            pl.BlockSpec((1, 4 * R, Q), lambda b: (b, 0, 0)),
            pl.BlockSpec((1, 2 * R, Q), lambda b: (b, 0, 0)),
            pl.BlockSpec((1, 4 * R, Q), lambda b: (b, 0, 0)),
            pl.BlockSpec((1, NUM_CLASSES * R, Q), lambda b: (b, 0, 0)),
        ],
        out_specs=[
            pl.BlockSpec((1, R, Q), lambda b: (b, 0, 0)),
            pl.BlockSpec((1, R, Q), lambda b: (b, 0, 0)),
            pl.BlockSpec((1, R, Q), lambda b: (b, 0, 0)),
            pl.BlockSpec((1, R, Q), lambda b: (b, 0, 0)),
            pl.BlockSpec((1, 1, 128), lambda b: (b, 0, 0)),
        ],
        out_shape=[row, row, row, row, stats_s],
    )(targets, pri_t, loc_t, conf_t, locr_t, confr_t)

    cem1 = cem1.reshape(BATCH, P)
    cemr = cemr.reshape(BATCH, P)
    cer = cer.reshape(BATCH, P)
    posr = posr.reshape(BATCH, P)
    stats = stats.reshape(BATCH, 128)

    sc = jax.ShapeDtypeStruct((1, 1), jnp.float32)
    o1, o2, o3, o4 = pl.pallas_call(
        _minek,
        out_shape=[sc, sc, sc, sc],
    )(cem1, cemr, cer, posr, stats)

    return (o1.reshape(()), o2.reshape(()), o3.reshape(()), o4.reshape(()))
